# Initial kernel scaffold; baseline (speedup 1.0000x reference)
import jax
import jax.numpy as jnp
from jax.experimental import pallas as pl


def kernel(feats, edge_index, graph_ids, W1, b1, Wr1, br1, g1, be1, W2, b2, Wr2, br2, g2, be2, Ww, bw, Wc1, bc1, gc1, bec1, Wc2, bc2):
    raise NotImplementedError("write your pallas kernel here")


# speedup vs baseline: 6.0269x; 6.0269x over previous
import functools

import jax
import jax.numpy as jnp
from jax import lax
from jax.experimental import pallas as pl
from jax.experimental.pallas import tpu as pltpu
from jax.experimental.pallas import tpu_sc as plsc

N = 10000
E = 320000
D = 128
B = 64
NT = 12
CH = 128


_CHUNK = 128
_NCHUNK = E // _CHUNK
_NW = 32
_FULL = _NCHUNK // _NW
_REM = _NCHUNK - _FULL * _NW


def _make_segsum():
    mesh = plsc.VectorSubcoreMesh(core_axis_name="c", subcore_axis_name="s")
    _RA = 624
    _RB = N - 15 * _RA

    @functools.partial(
        pl.kernel,
        mesh=mesh,
        out_type=jax.ShapeDtypeStruct((2, N, D), jnp.float32),
        scratch_types=[
            pltpu.VMEM((_CHUNK,), jnp.int32),
            pltpu.VMEM((_CHUNK,), jnp.int32),
            pltpu.VMEM((_CHUNK, D), jnp.float32),
            pltpu.VMEM_SHARED((N, D), jnp.float32),
            pltpu.SemaphoreType.DMA,
        ],
    )
    def segsum(x_hbm, src_hbm, dst_hbm, zeros_hbm, out_hbm,
               src_v, dst_v, rows_v, agg_sh, sem):
        cid = lax.axis_index("c")
        sid = lax.axis_index("s")
        wid = sid * 2 + cid

        r0 = sid * _RA

        @pl.when(sid < 15)
        def _():
            pltpu.sync_copy(zeros_hbm.at[pl.ds(r0, _RA)],
                            agg_sh.at[pl.ds(r0, _RA)])

        @pl.when(sid == 15)
        def _():
            pltpu.sync_copy(zeros_hbm.at[pl.ds(15 * _RA, _RB)],
                            agg_sh.at[pl.ds(15 * _RA, _RB)])

        plsc.subcore_barrier()

        def do_chunk(c):
            base = c * _CHUNK
            pltpu.sync_copy(src_hbm.at[pl.ds(base, _CHUNK)], src_v)
            pltpu.sync_copy(dst_hbm.at[pl.ds(base, _CHUNK)], dst_v)
            pltpu.async_copy(x_hbm.at[src_v], rows_v, sem).wait()
            pltpu.sync_copy(rows_v, agg_sh.at[dst_v], add=True)

        def body(i, carry):
            do_chunk(wid + i * _NW)
            return carry

        lax.fori_loop(0, _FULL, body, 0)

        @pl.when(wid < _REM)
        def _():
            do_chunk(_FULL * _NW + wid)

        plsc.subcore_barrier()

        @pl.when(sid < 15)
        def _():
            pltpu.sync_copy(agg_sh.at[pl.ds(r0, _RA)],
                            out_hbm.at[cid, pl.ds(r0, _RA)])

        @pl.when(sid == 15)
        def _():
            pltpu.sync_copy(agg_sh.at[pl.ds(15 * _RA, _RB)],
                            out_hbm.at[cid, pl.ds(15 * _RA, _RB)])

    return segsum


_segsum = _make_segsum()


_BLK = 1000
_NBLK = N // _BLK


def _dense_body(p0_ref, p1_ref, x_ref, w_ref, b_ref, wr_ref, br_ref,
                h_ref, s_ref, ss_ref):
    i = pl.program_id(0)
    agg = p0_ref[...] + p1_ref[...]
    h = jnp.maximum(jnp.dot(agg, w_ref[...],
                            preferred_element_type=jnp.float32) + b_ref[...], 0.0)
    res = jnp.maximum(jnp.dot(x_ref[...], wr_ref[...],
                              preferred_element_type=jnp.float32) + br_ref[...], 0.0)
    h = h + res
    h_ref[...] = h

    @pl.when(i == 0)
    def _():
        s_ref[...] = jnp.zeros_like(s_ref)
        ss_ref[...] = jnp.zeros_like(ss_ref)

    s_ref[...] += jnp.sum(h, axis=0, keepdims=True)
    ss_ref[...] += jnp.sum(h * h, axis=0, keepdims=True)


def _dense_layer(parts, x, w, b, wr, br):
    grid = (_NBLK,)
    row_spec = pl.BlockSpec((_BLK, D), lambda i: (i, 0))
    mat_spec = pl.BlockSpec((D, D), lambda i: (0, 0))
    vec_spec = pl.BlockSpec((1, D), lambda i: (0, 0))
    return pl.pallas_call(
        _dense_body,
        grid=grid,
        in_specs=[row_spec, row_spec, row_spec, mat_spec, vec_spec,
                  mat_spec, vec_spec],
        out_specs=[row_spec, vec_spec, vec_spec],
        out_shape=[
            jax.ShapeDtypeStruct((N, D), jnp.float32),
            jax.ShapeDtypeStruct((1, D), jnp.float32),
            jax.ShapeDtypeStruct((1, D), jnp.float32),
        ],
    )(parts[0], parts[1], x, w, b.reshape(1, D), wr, br.reshape(1, D))


def _norm_body(h_ref, s_ref, ss_ref, g_ref, be_ref, o_ref):
    mean = s_ref[...] / N
    var = ss_ref[...] / N - mean * mean
    inv = lax.rsqrt(var + 1e-5)
    o_ref[...] = (h_ref[...] - mean) * inv * g_ref[...] + be_ref[...]


def _normalize(h, s, ss, g, be):
    row_spec = pl.BlockSpec((_BLK, D), lambda i: (i, 0))
    vec_spec = pl.BlockSpec((1, D), lambda i: (0, 0))
    return pl.pallas_call(
        _norm_body,
        grid=(_NBLK,),
        in_specs=[row_spec, vec_spec, vec_spec, vec_spec, vec_spec],
        out_specs=row_spec,
        out_shape=jax.ShapeDtypeStruct((N, D), jnp.float32),
    )(h, s, ss, g.reshape(1, D), be.reshape(1, D))


def _pool_body(h_ref, s_ref, ss_ref, g_ref, be_ref, gidr_ref, gidc_ref,
               ww_ref, bw_ref, hsum_ref, hmax_ref):
    i = pl.program_id(0)
    mean = s_ref[...] / N
    var = ss_ref[...] / N - mean * mean
    inv = lax.rsqrt(var + 1e-5)
    x = (h_ref[...] - mean) * inv * g_ref[...] + be_ref[...]

    wcol = jax.nn.sigmoid(jnp.dot(x, ww_ref[...],
                                  preferred_element_type=jnp.float32)
                          + bw_ref[...])
    xw = x * wcol

    gidr = gidr_ref[0]
    onehot = (lax.broadcasted_iota(jnp.int32, (B, _BLK), 0) == gidr
              ).astype(jnp.float32)

    @pl.when(i == 0)
    def _():
        hsum_ref[...] = jnp.zeros_like(hsum_ref)
        hmax_ref[...] = jnp.full_like(hmax_ref, -jnp.inf)

    hsum_ref[...] += jnp.dot(onehot, xw, preferred_element_type=jnp.float32)

    gidc = gidc_ref[...]
    lo = gidr[0, 0]
    hi = gidr[0, _BLK - 1]

    def mbody(bseg, carry):
        m = jnp.max(jnp.where(gidc == bseg, x, -jnp.inf), axis=0,
                    keepdims=True)
        hmax_ref[pl.ds(bseg, 1), :] = jnp.maximum(hmax_ref[pl.ds(bseg, 1), :], m)
        return carry

    lax.fori_loop(lo, hi + 1, mbody, 0)


def _pool(h, s, ss, g, be, gid_row, gid_col, ww, bw):
    row_spec = pl.BlockSpec((_BLK, D), lambda i: (i, 0))
    vec_spec = pl.BlockSpec((1, D), lambda i: (0, 0))
    out_spec = pl.BlockSpec((B, D), lambda i: (0, 0))
    return pl.pallas_call(
        _pool_body,
        grid=(_NBLK,),
        in_specs=[row_spec, vec_spec, vec_spec, vec_spec, vec_spec,
                  pl.BlockSpec((1, 1, _BLK), lambda i: (i, 0, 0)),
                  pl.BlockSpec((_BLK, 1), lambda i: (i, 0)),
                  pl.BlockSpec((D, 1), lambda i: (0, 0)),
                  pl.BlockSpec((1, 1), lambda i: (0, 0))],
        out_specs=[out_spec, out_spec],
        out_shape=[
            jax.ShapeDtypeStruct((B, D), jnp.float32),
            jax.ShapeDtypeStruct((B, D), jnp.float32),
        ],
    )(h, s, ss, g.reshape(1, D), be.reshape(1, D),
      gid_row, gid_col, ww, bw.reshape(1, 1))


def _cls_body(hs_ref, hm_ref, wc1_ref, bc1_ref, gc1_ref, bec1_ref,
              wc2_ref, bc2_ref, o_ref):
    z = (jnp.dot(hs_ref[...], wc1_ref[0:D, :],
                 preferred_element_type=jnp.float32)
         + jnp.dot(hm_ref[...], wc1_ref[D:2 * D, :],
                   preferred_element_type=jnp.float32)
         + bc1_ref[...])
    z = jnp.maximum(z, 0.0)
    mean = jnp.mean(z, axis=0, keepdims=True)
    zc = z - mean
    var = jnp.mean(zc * zc, axis=0, keepdims=True)
    zn = zc * lax.rsqrt(var + 1e-5) * gc1_ref[...] + bec1_ref[...]
    o_ref[...] = jnp.dot(zn, wc2_ref[...],
                         preferred_element_type=jnp.float32) + bc2_ref[...]


def _classifier(hs, hm, wc1, bc1, gc1, bec1, wc2, bc2):
    return pl.pallas_call(
        _cls_body,
        out_shape=jax.ShapeDtypeStruct((B, NT), jnp.float32),
    )(hs, hm, wc1, bc1.reshape(1, CH), gc1.reshape(1, CH),
      bec1.reshape(1, CH), wc2, bc2.reshape(1, NT))


def kernel(feats, edge_index, graph_ids, W1, b1, Wr1, br1, g1, be1,
           W2, b2, Wr2, br2, g2, be2, Ww, bw, Wc1, bc1, gc1, bec1, Wc2, bc2):
    src = edge_index[0]
    dst = edge_index[1]
    zeros = jnp.zeros((N, D), jnp.float32)
    gid_row = graph_ids.reshape(_NBLK, 1, _BLK)
    gid_col = graph_ids.reshape(N, 1)

    parts1 = _segsum(feats, src, dst, zeros)
    h1, s1, ss1 = _dense_layer(parts1, feats, W1, b1, Wr1, br1)
    x1 = _normalize(h1, s1, ss1, g1, be1)

    parts2 = _segsum(x1, src, dst, zeros)
    h2, s2, ss2 = _dense_layer(parts2, x1, W2, b2, Wr2, br2)

    hs, hm = _pool(h2, s2, ss2, g2, be2, gid_row, gid_col, Ww, bw)
    return _classifier(hs, hm, Wc1, bc1, gc1, bec1, Wc2, bc2)


# speedup vs baseline: 9.6622x; 1.6032x over previous
import functools

import jax
import jax.numpy as jnp
from jax import lax
from jax.experimental import pallas as pl
from jax.experimental.pallas import tpu as pltpu
from jax.experimental.pallas import tpu_sc as plsc

N = 10000
E = 320000
D = 128
B = 64
NT = 12
CH = 128


_CHUNK = 128
_NW = 32
_CPW = 80
_HALF = 40
_E2 = _NW * _CPW * _CHUNK
_NJUNK = 8


def _make_segsum():
    mesh = plsc.VectorSubcoreMesh(core_axis_name="c", subcore_axis_name="s")
    _RA = 624
    _RB = N - 15 * _RA

    @functools.partial(
        pl.kernel,
        mesh=mesh,
        out_type=jax.ShapeDtypeStruct((2, N, D), jnp.float32),
        scratch_types=[
            pltpu.VMEM((_HALF * _CHUNK,), jnp.int32),
            pltpu.VMEM((_HALF, _CHUNK), jnp.int32),
            pltpu.VMEM((_CHUNK, D), jnp.float32),
            pltpu.VMEM((_CHUNK, D), jnp.float32),
            pltpu.VMEM_SHARED((N + _NJUNK, D), jnp.float32),
            pltpu.SemaphoreType.DMA,
            pltpu.SemaphoreType.DMA,
        ],
    )
    def segsum(x_hbm, src_hbm, dst_hbm, zeros_hbm, out_hbm,
               src_v, dst_v, rows0, rows1, agg_sh, gsem0, gsem1):
        cid = lax.axis_index("c")
        sid = lax.axis_index("s")
        wid = sid * 2 + cid

        r0 = sid * _RA

        @pl.when(sid < 15)
        def _():
            pltpu.sync_copy(zeros_hbm.at[pl.ds(r0, _RA)],
                            agg_sh.at[pl.ds(r0, _RA)])

        @pl.when(sid == 15)
        def _():
            pltpu.sync_copy(zeros_hbm.at[pl.ds(15 * _RA, _RB)],
                            agg_sh.at[pl.ds(15 * _RA, _RB)])

        plsc.subcore_barrier()

        def sidx(j):
            return src_v.at[pl.ds(j * _CHUNK, _CHUNK)]

        def g_start(j, buf):
            pltpu.async_copy(x_hbm.at[sidx(j)], buf, gsem0)

        def g_wait(j, buf):
            pltpu.make_async_copy(x_hbm.at[sidx(j)], buf, gsem0).wait()

        def s_start(j, buf):
            pltpu.async_copy(buf, agg_sh.at[dst_v.at[j]], gsem1, add=True)

        def s_wait(j, buf):
            pltpu.make_async_copy(buf, agg_sh.at[dst_v.at[j]], gsem1).wait()

        def body(i, carry):
            j0 = 2 * i
            j1 = j0 + 1
            g_wait(j0, rows0)

            @pl.when(i > 0)
            def _():
                s_wait(j0 - 1, rows1)

            g_start(j1, rows1)
            s_start(j0, rows0)
            g_wait(j1, rows1)
            s_wait(j0, rows0)

            @pl.when(j1 + 1 < _HALF)
            def _():
                g_start(j1 + 1, rows0)

            s_start(j1, rows1)
            return carry

        for h in range(_CPW // _HALF):
            pltpu.sync_copy(
                src_hbm.at[pl.ds((wid * _CPW + h * _HALF) * _CHUNK,
                                 _HALF * _CHUNK)], src_v)
            pltpu.sync_copy(dst_hbm.at[pl.ds(wid * _CPW + h * _HALF, _HALF)],
                            dst_v)
            g_start(0, rows0)
            lax.fori_loop(0, _HALF // 2, body, 0)
            s_wait(_HALF - 1, rows1)

        plsc.subcore_barrier()

        @pl.when(sid < 15)
        def _():
            pltpu.sync_copy(agg_sh.at[pl.ds(r0, _RA)],
                            out_hbm.at[cid, pl.ds(r0, _RA)])

        @pl.when(sid == 15)
        def _():
            pltpu.sync_copy(agg_sh.at[pl.ds(15 * _RA, _RB)],
                            out_hbm.at[cid, pl.ds(15 * _RA, _RB)])

    return segsum


_segsum = _make_segsum()


_BLK = 1000
_NBLK = N // _BLK


def _dense_body(p0_ref, p1_ref, x_ref, w_ref, b_ref, wr_ref, br_ref,
                h_ref, s_ref, ss_ref):
    i = pl.program_id(0)
    agg = p0_ref[...] + p1_ref[...]
    h = jnp.maximum(jnp.dot(agg, w_ref[...],
                            preferred_element_type=jnp.float32) + b_ref[...], 0.0)
    res = jnp.maximum(jnp.dot(x_ref[...], wr_ref[...],
                              preferred_element_type=jnp.float32) + br_ref[...], 0.0)
    h = h + res
    h_ref[...] = h

    @pl.when(i == 0)
    def _():
        s_ref[...] = jnp.zeros_like(s_ref)
        ss_ref[...] = jnp.zeros_like(ss_ref)

    s_ref[...] += jnp.sum(h, axis=0, keepdims=True)
    ss_ref[...] += jnp.sum(h * h, axis=0, keepdims=True)


def _dense_layer(parts, x, w, b, wr, br):
    grid = (_NBLK,)
    row_spec = pl.BlockSpec((_BLK, D), lambda i: (i, 0))
    mat_spec = pl.BlockSpec((D, D), lambda i: (0, 0))
    vec_spec = pl.BlockSpec((1, D), lambda i: (0, 0))
    return pl.pallas_call(
        _dense_body,
        grid=grid,
        in_specs=[row_spec, row_spec, row_spec, mat_spec, vec_spec,
                  mat_spec, vec_spec],
        out_specs=[row_spec, vec_spec, vec_spec],
        out_shape=[
            jax.ShapeDtypeStruct((N, D), jnp.float32),
            jax.ShapeDtypeStruct((1, D), jnp.float32),
            jax.ShapeDtypeStruct((1, D), jnp.float32),
        ],
    )(parts[0], parts[1], x, w, b.reshape(1, D), wr, br.reshape(1, D))


def _norm_body(h_ref, s_ref, ss_ref, g_ref, be_ref, o_ref):
    mean = s_ref[...] / N
    var = ss_ref[...] / N - mean * mean
    inv = lax.rsqrt(var + 1e-5)
    o_ref[...] = (h_ref[...] - mean) * inv * g_ref[...] + be_ref[...]


def _normalize(h, s, ss, g, be):
    row_spec = pl.BlockSpec((_BLK, D), lambda i: (i, 0))
    vec_spec = pl.BlockSpec((1, D), lambda i: (0, 0))
    return pl.pallas_call(
        _norm_body,
        grid=(_NBLK,),
        in_specs=[row_spec, vec_spec, vec_spec, vec_spec, vec_spec],
        out_specs=row_spec,
        out_shape=jax.ShapeDtypeStruct((N, D), jnp.float32),
    )(h, s, ss, g.reshape(1, D), be.reshape(1, D))


def _pool_body(h_ref, s_ref, ss_ref, g_ref, be_ref, gidr_ref, gidc_ref,
               ww_ref, bw_ref, hsum_ref, hmax_ref):
    i = pl.program_id(0)
    mean = s_ref[...] / N
    var = ss_ref[...] / N - mean * mean
    inv = lax.rsqrt(var + 1e-5)
    x = (h_ref[...] - mean) * inv * g_ref[...] + be_ref[...]

    wcol = jax.nn.sigmoid(jnp.dot(x, ww_ref[...],
                                  preferred_element_type=jnp.float32)
                          + bw_ref[...])
    xw = x * wcol

    gidr = gidr_ref[0]
    onehot = (lax.broadcasted_iota(jnp.int32, (B, _BLK), 0) == gidr
              ).astype(jnp.float32)

    @pl.when(i == 0)
    def _():
        hsum_ref[...] = jnp.zeros_like(hsum_ref)
        hmax_ref[...] = jnp.full_like(hmax_ref, -jnp.inf)

    hsum_ref[...] += jnp.dot(onehot, xw, preferred_element_type=jnp.float32)

    gidc = gidc_ref[...]
    lo = gidr[0, 0]
    hi = gidr[0, _BLK - 1]

    def mbody(bseg, carry):
        m = jnp.max(jnp.where(gidc == bseg, x, -jnp.inf), axis=0,
                    keepdims=True)
        hmax_ref[pl.ds(bseg, 1), :] = jnp.maximum(hmax_ref[pl.ds(bseg, 1), :], m)
        return carry

    lax.fori_loop(lo, hi + 1, mbody, 0)


def _pool(h, s, ss, g, be, gid_row, gid_col, ww, bw):
    row_spec = pl.BlockSpec((_BLK, D), lambda i: (i, 0))
    vec_spec = pl.BlockSpec((1, D), lambda i: (0, 0))
    out_spec = pl.BlockSpec((B, D), lambda i: (0, 0))
    return pl.pallas_call(
        _pool_body,
        grid=(_NBLK,),
        in_specs=[row_spec, vec_spec, vec_spec, vec_spec, vec_spec,
                  pl.BlockSpec((1, 1, _BLK), lambda i: (i, 0, 0)),
                  pl.BlockSpec((_BLK, 1), lambda i: (i, 0)),
                  pl.BlockSpec((D, 1), lambda i: (0, 0)),
                  pl.BlockSpec((1, 1), lambda i: (0, 0))],
        out_specs=[out_spec, out_spec],
        out_shape=[
            jax.ShapeDtypeStruct((B, D), jnp.float32),
            jax.ShapeDtypeStruct((B, D), jnp.float32),
        ],
    )(h, s, ss, g.reshape(1, D), be.reshape(1, D),
      gid_row, gid_col, ww, bw.reshape(1, 1))


def _cls_body(hs_ref, hm_ref, wc1_ref, bc1_ref, gc1_ref, bec1_ref,
              wc2_ref, bc2_ref, o_ref):
    z = (jnp.dot(hs_ref[...], wc1_ref[0:D, :],
                 preferred_element_type=jnp.float32)
         + jnp.dot(hm_ref[...], wc1_ref[D:2 * D, :],
                   preferred_element_type=jnp.float32)
         + bc1_ref[...])
    z = jnp.maximum(z, 0.0)
    mean = jnp.mean(z, axis=0, keepdims=True)
    zc = z - mean
    var = jnp.mean(zc * zc, axis=0, keepdims=True)
    zn = zc * lax.rsqrt(var + 1e-5) * gc1_ref[...] + bec1_ref[...]
    o_ref[...] = jnp.dot(zn, wc2_ref[...],
                         preferred_element_type=jnp.float32) + bc2_ref[...]


def _classifier(hs, hm, wc1, bc1, gc1, bec1, wc2, bc2):
    return pl.pallas_call(
        _cls_body,
        out_shape=jax.ShapeDtypeStruct((B, NT), jnp.float32),
    )(hs, hm, wc1, bc1.reshape(1, CH), gc1.reshape(1, CH),
      bec1.reshape(1, CH), wc2, bc2.reshape(1, NT))


def kernel(feats, edge_index, graph_ids, W1, b1, Wr1, br1, g1, be1,
           W2, b2, Wr2, br2, g2, be2, Ww, bw, Wc1, bc1, gc1, bec1, Wc2, bc2):
    pad = _E2 - E
    pad_idx = jnp.arange(pad, dtype=jnp.int32)
    src = jnp.concatenate([edge_index[0], (pad_idx * 7) % N])
    dst = jnp.concatenate([edge_index[1], N + (pad_idx % _NJUNK)]).reshape(
        _NW * _CPW, _CHUNK)
    zeros = jnp.zeros((N, D), jnp.float32)
    gid_row = graph_ids.reshape(_NBLK, 1, _BLK)
    gid_col = graph_ids.reshape(N, 1)

    parts1 = _segsum(feats, src, dst, zeros)
    h1, s1, ss1 = _dense_layer(parts1, feats, W1, b1, Wr1, br1)
    x1 = _normalize(h1, s1, ss1, g1, be1)

    parts2 = _segsum(x1, src, dst, zeros)
    h2, s2, ss2 = _dense_layer(parts2, x1, W2, b2, Wr2, br2)

    hs, hm = _pool(h2, s2, ss2, g2, be2, gid_row, gid_col, Ww, bw)
    return _classifier(hs, hm, Wc1, bc1, gc1, bec1, Wc2, bc2)


# speedup vs baseline: 10.0979x; 1.0451x over previous
import functools

import jax
import jax.numpy as jnp
from jax import lax
from jax.experimental import pallas as pl
from jax.experimental.pallas import tpu as pltpu
from jax.experimental.pallas import tpu_sc as plsc

N = 10000
E = 320000
D = 128
B = 64
NT = 12
CH = 128


_CHUNK = 64
_NW = 32
_CPW = 160
_HALF = 80
_E2 = _NW * _CPW * _CHUNK
_NJUNK = 8


def _make_segsum():
    mesh = plsc.VectorSubcoreMesh(core_axis_name="c", subcore_axis_name="s")
    _RA = 624
    _RB = N - 15 * _RA

    @functools.partial(
        pl.kernel,
        mesh=mesh,
        out_type=jax.ShapeDtypeStruct((2, N, D), jnp.float32),
        scratch_types=[
            pltpu.VMEM((_HALF * _CHUNK,), jnp.int32),
            pltpu.VMEM((_HALF, _CHUNK), jnp.int32),
            pltpu.VMEM((_CHUNK, D), jnp.float32),
            pltpu.VMEM((_CHUNK, D), jnp.float32),
            pltpu.VMEM((_CHUNK, D), jnp.float32),
            pltpu.VMEM((_CHUNK, D), jnp.float32),
            pltpu.VMEM_SHARED((N + _NJUNK, D), jnp.float32),
            pltpu.SemaphoreType.DMA,
            pltpu.SemaphoreType.DMA,
            pltpu.SemaphoreType.DMA,
            pltpu.SemaphoreType.DMA,
        ],
    )
    def segsum(x_hbm, src_hbm, dst_hbm, zeros_hbm, out_hbm,
               src_v, dst_v, rows0, rows1, rows2, rows3,
               agg_sh, gsem0, gsem1, ssem0, ssem1):
        cid = lax.axis_index("c")
        sid = lax.axis_index("s")
        wid = sid * 2 + cid

        r0 = sid * _RA

        @pl.when(sid < 15)
        def _():
            pltpu.sync_copy(zeros_hbm.at[pl.ds(r0, _RA)],
                            agg_sh.at[pl.ds(r0, _RA)])

        @pl.when(sid == 15)
        def _():
            pltpu.sync_copy(zeros_hbm.at[pl.ds(15 * _RA, _RB)],
                            agg_sh.at[pl.ds(15 * _RA, _RB)])

        plsc.subcore_barrier()

        def sidx(j):
            return src_v.at[pl.ds(j * _CHUNK, _CHUNK)]

        bufs = (rows0, rows1, rows2, rows3)
        gsems = (gsem0, gsem1)
        ssems = (ssem0, ssem1)

        def g_start(j, k):
            pltpu.async_copy(x_hbm.at[sidx(j)], bufs[k], gsems[k & 1])

        def g_wait(j, k):
            pltpu.make_async_copy(x_hbm.at[sidx(j)], bufs[k],
                                  gsems[k & 1]).wait()

        def s_start(j, k):
            pltpu.async_copy(bufs[k], agg_sh.at[dst_v.at[j]], ssems[k & 1],
                             add=True)

        def s_wait(j, k):
            pltpu.make_async_copy(bufs[k], agg_sh.at[dst_v.at[j]],
                                  ssems[k & 1]).wait()

        def body(i, carry):
            for k in range(4):
                j = 4 * i + k
                g_wait(j, k)
                if k < 2:
                    @pl.when(i > 0)
                    def _():
                        s_wait(j - 2, (k + 2) % 4)
                else:
                    s_wait(j - 2, (k + 2) % 4)
                s_start(j, k)
                if k < 2:
                    g_start(j + 2, (k + 2) % 4)
                else:
                    @pl.when(j + 2 < _HALF)
                    def _():
                        g_start(j + 2, (k + 2) % 4)
            return carry

        for h in range(_CPW // _HALF):
            pltpu.sync_copy(
                src_hbm.at[pl.ds((wid * _CPW + h * _HALF) * _CHUNK,
                                 _HALF * _CHUNK)], src_v)
            pltpu.sync_copy(dst_hbm.at[pl.ds(wid * _CPW + h * _HALF, _HALF)],
                            dst_v)
            g_start(0, 0)
            g_start(1, 1)
            lax.fori_loop(0, _HALF // 4, body, 0)
            s_wait(_HALF - 2, 2)
            s_wait(_HALF - 1, 3)

        plsc.subcore_barrier()

        @pl.when(sid < 15)
        def _():
            pltpu.sync_copy(agg_sh.at[pl.ds(r0, _RA)],
                            out_hbm.at[cid, pl.ds(r0, _RA)])

        @pl.when(sid == 15)
        def _():
            pltpu.sync_copy(agg_sh.at[pl.ds(15 * _RA, _RB)],
                            out_hbm.at[cid, pl.ds(15 * _RA, _RB)])

    return segsum


_segsum = _make_segsum()


_BLK = 1000
_NBLK = N // _BLK


def _dense_body(p0_ref, p1_ref, x_ref, w_ref, b_ref, wr_ref, br_ref,
                h_ref, s_ref, ss_ref):
    i = pl.program_id(0)
    agg = p0_ref[...] + p1_ref[...]
    h = jnp.maximum(jnp.dot(agg, w_ref[...],
                            preferred_element_type=jnp.float32) + b_ref[...], 0.0)
    res = jnp.maximum(jnp.dot(x_ref[...], wr_ref[...],
                              preferred_element_type=jnp.float32) + br_ref[...], 0.0)
    h = h + res
    h_ref[...] = h

    @pl.when(i == 0)
    def _():
        s_ref[...] = jnp.zeros_like(s_ref)
        ss_ref[...] = jnp.zeros_like(ss_ref)

    s_ref[...] += jnp.sum(h, axis=0, keepdims=True)
    ss_ref[...] += jnp.sum(h * h, axis=0, keepdims=True)


def _dense_layer(parts, x, w, b, wr, br):
    grid = (_NBLK,)
    row_spec = pl.BlockSpec((_BLK, D), lambda i: (i, 0))
    mat_spec = pl.BlockSpec((D, D), lambda i: (0, 0))
    vec_spec = pl.BlockSpec((1, D), lambda i: (0, 0))
    return pl.pallas_call(
        _dense_body,
        grid=grid,
        in_specs=[row_spec, row_spec, row_spec, mat_spec, vec_spec,
                  mat_spec, vec_spec],
        out_specs=[row_spec, vec_spec, vec_spec],
        out_shape=[
            jax.ShapeDtypeStruct((N, D), jnp.float32),
            jax.ShapeDtypeStruct((1, D), jnp.float32),
            jax.ShapeDtypeStruct((1, D), jnp.float32),
        ],
    )(parts[0], parts[1], x, w, b.reshape(1, D), wr, br.reshape(1, D))


def _norm_body(h_ref, s_ref, ss_ref, g_ref, be_ref, o_ref):
    mean = s_ref[...] / N
    var = ss_ref[...] / N - mean * mean
    inv = lax.rsqrt(var + 1e-5)
    o_ref[...] = (h_ref[...] - mean) * inv * g_ref[...] + be_ref[...]


def _normalize(h, s, ss, g, be):
    row_spec = pl.BlockSpec((_BLK, D), lambda i: (i, 0))
    vec_spec = pl.BlockSpec((1, D), lambda i: (0, 0))
    return pl.pallas_call(
        _norm_body,
        grid=(_NBLK,),
        in_specs=[row_spec, vec_spec, vec_spec, vec_spec, vec_spec],
        out_specs=row_spec,
        out_shape=jax.ShapeDtypeStruct((N, D), jnp.float32),
    )(h, s, ss, g.reshape(1, D), be.reshape(1, D))


def _pool_body(h_ref, s_ref, ss_ref, g_ref, be_ref, gidr_ref, gidc_ref,
               ww_ref, bw_ref, hsum_ref, hmax_ref):
    i = pl.program_id(0)
    mean = s_ref[...] / N
    var = ss_ref[...] / N - mean * mean
    inv = lax.rsqrt(var + 1e-5)
    x = (h_ref[...] - mean) * inv * g_ref[...] + be_ref[...]

    wcol = jax.nn.sigmoid(jnp.dot(x, ww_ref[...],
                                  preferred_element_type=jnp.float32)
                          + bw_ref[...])
    xw = x * wcol

    gidr = gidr_ref[0]
    onehot = (lax.broadcasted_iota(jnp.int32, (B, _BLK), 0) == gidr
              ).astype(jnp.float32)

    @pl.when(i == 0)
    def _():
        hsum_ref[...] = jnp.zeros_like(hsum_ref)
        hmax_ref[...] = jnp.full_like(hmax_ref, -jnp.inf)

    hsum_ref[...] += jnp.dot(onehot, xw, preferred_element_type=jnp.float32)

    gidc = gidc_ref[...]
    lo = gidr[0, 0]
    hi = gidr[0, _BLK - 1]

    def mbody(bseg, carry):
        m = jnp.max(jnp.where(gidc == bseg, x, -jnp.inf), axis=0,
                    keepdims=True)
        hmax_ref[pl.ds(bseg, 1), :] = jnp.maximum(hmax_ref[pl.ds(bseg, 1), :], m)
        return carry

    lax.fori_loop(lo, hi + 1, mbody, 0)


def _pool(h, s, ss, g, be, gid_row, gid_col, ww, bw):
    row_spec = pl.BlockSpec((_BLK, D), lambda i: (i, 0))
    vec_spec = pl.BlockSpec((1, D), lambda i: (0, 0))
    out_spec = pl.BlockSpec((B, D), lambda i: (0, 0))
    return pl.pallas_call(
        _pool_body,
        grid=(_NBLK,),
        in_specs=[row_spec, vec_spec, vec_spec, vec_spec, vec_spec,
                  pl.BlockSpec((1, 1, _BLK), lambda i: (i, 0, 0)),
                  pl.BlockSpec((_BLK, 1), lambda i: (i, 0)),
                  pl.BlockSpec((D, 1), lambda i: (0, 0)),
                  pl.BlockSpec((1, 1), lambda i: (0, 0))],
        out_specs=[out_spec, out_spec],
        out_shape=[
            jax.ShapeDtypeStruct((B, D), jnp.float32),
            jax.ShapeDtypeStruct((B, D), jnp.float32),
        ],
    )(h, s, ss, g.reshape(1, D), be.reshape(1, D),
      gid_row, gid_col, ww, bw.reshape(1, 1))


def _cls_body(hs_ref, hm_ref, wc1_ref, bc1_ref, gc1_ref, bec1_ref,
              wc2_ref, bc2_ref, o_ref):
    z = (jnp.dot(hs_ref[...], wc1_ref[0:D, :],
                 preferred_element_type=jnp.float32)
         + jnp.dot(hm_ref[...], wc1_ref[D:2 * D, :],
                   preferred_element_type=jnp.float32)
         + bc1_ref[...])
    z = jnp.maximum(z, 0.0)
    mean = jnp.mean(z, axis=0, keepdims=True)
    zc = z - mean
    var = jnp.mean(zc * zc, axis=0, keepdims=True)
    zn = zc * lax.rsqrt(var + 1e-5) * gc1_ref[...] + bec1_ref[...]
    o_ref[...] = jnp.dot(zn, wc2_ref[...],
                         preferred_element_type=jnp.float32) + bc2_ref[...]


def _classifier(hs, hm, wc1, bc1, gc1, bec1, wc2, bc2):
    return pl.pallas_call(
        _cls_body,
        out_shape=jax.ShapeDtypeStruct((B, NT), jnp.float32),
    )(hs, hm, wc1, bc1.reshape(1, CH), gc1.reshape(1, CH),
      bec1.reshape(1, CH), wc2, bc2.reshape(1, NT))


def kernel(feats, edge_index, graph_ids, W1, b1, Wr1, br1, g1, be1,
           W2, b2, Wr2, br2, g2, be2, Ww, bw, Wc1, bc1, gc1, bec1, Wc2, bc2):
    pad = _E2 - E
    pad_idx = jnp.arange(pad, dtype=jnp.int32)
    src = jnp.concatenate([edge_index[0], (pad_idx * 7) % N])
    dst = jnp.concatenate([edge_index[1], N + (pad_idx % _NJUNK)]).reshape(
        _NW * _CPW, _CHUNK)
    zeros = jnp.zeros((N, D), jnp.float32)
    gid_row = graph_ids.reshape(_NBLK, 1, _BLK)
    gid_col = graph_ids.reshape(N, 1)

    parts1 = _segsum(feats, src, dst, zeros)
    h1, s1, ss1 = _dense_layer(parts1, feats, W1, b1, Wr1, br1)
    x1 = _normalize(h1, s1, ss1, g1, be1)

    parts2 = _segsum(x1, src, dst, zeros)
    h2, s2, ss2 = _dense_layer(parts2, x1, W2, b2, Wr2, br2)

    hs, hm = _pool(h2, s2, ss2, g2, be2, gid_row, gid_col, Ww, bw)
    return _classifier(hs, hm, Wc1, bc1, gc1, bec1, Wc2, bc2)
